# baseline (speedup 1.0000x reference)
import jax
import jax.numpy as jnp
from jax import lax
from jax.experimental import pallas as pl
from jax.experimental.pallas import tpu as pltpu
from jax.experimental.pallas import tpu_sc as plsc

_B, _S, _L, _D = 1024, 20, 4, 768
_VOCAB = 102
_T = _B * _S
_NC, _NS = 2, 16
_NW = _NC * _NS
_TPW = _T // _NW
_CHUNK = 16
_NCHUNK = _TPW // _CHUNK


def _body(ids_hbm, table_hbm, out_hbm, table_v, ids_v, obuf, sem0, sem1):
    wid = lax.axis_index("s") * _NC + lax.axis_index("c")
    base = wid * _TPW
    pltpu.sync_copy(table_hbm, table_v)
    pltpu.sync_copy(ids_hbm.at[pl.ds(base * _L, _TPW * _L)], ids_v)
    zero = jnp.zeros((16,), jnp.float32)
    for j in range(_D // 16):
        table_v[0, pl.ds(j * 16, 16)] = zero
    sems = (sem0, sem1)

    def pair_body(c2, carry):
        for b in range(2):
            c = c2 * 2 + b

            @pl.when(c2 > 0)
            def _wait():
                pltpu.make_async_copy(
                    obuf.at[b], out_hbm.at[pl.ds(base, _CHUNK)], sems[b]).wait()

            def grp_body(g, carry2):
                iv = ids_v[pl.ds((c * _CHUNK + g * 4) * _L, 16)]
                for tt in range(4):
                    i0 = iv[4 * tt + 0]
                    i1 = iv[4 * tt + 1]
                    i2 = iv[4 * tt + 2]
                    i3 = iv[4 * tt + 3]
                    t = g * 4 + tt
                    for j in range(_D // 16):
                        ds = pl.ds(j * 16, 16)
                        v = jnp.maximum(
                            jnp.maximum(table_v[i0, ds], table_v[i1, ds]),
                            jnp.maximum(table_v[i2, ds], table_v[i3, ds]))
                        obuf[b, t, ds] = v
                return carry2

            lax.fori_loop(0, _CHUNK // 4, grp_body, 0)
            pltpu.async_copy(
                obuf.at[b], out_hbm.at[pl.ds(base + c * _CHUNK, _CHUNK)],
                sems[b])
        return carry

    lax.fori_loop(0, _NCHUNK // 2, pair_body, 0)
    for b in range(2):
        pltpu.make_async_copy(
            obuf.at[b], out_hbm.at[pl.ds(base, _CHUNK)], sems[b]).wait()


@jax.jit
def _glyph(ids_flat, table):
    mesh = plsc.VectorSubcoreMesh(core_axis_name="c", subcore_axis_name="s")
    f = pl.kernel(
        _body,
        out_type=jax.ShapeDtypeStruct((_T, _D), jnp.float32),
        mesh=mesh,
        scratch_types=[
            pltpu.VMEM((_VOCAB, _D), jnp.float32),
            pltpu.VMEM((_TPW * _L,), jnp.int32),
            pltpu.VMEM((2, _CHUNK, _D), jnp.float32),
            pltpu.SemaphoreType.DMA,
            pltpu.SemaphoreType.DMA,
        ],
    )
    return f(ids_flat, table)


def kernel(zixing_ids, table):
    ids_flat = zixing_ids.reshape(_T * _L)
    out = _glyph(ids_flat, table)
    return out.reshape(_B, _S, _D)


# speedup vs baseline: 1.8645x; 1.8645x over previous
import jax
import jax.numpy as jnp
from jax import lax
from jax.experimental import pallas as pl
from jax.experimental.pallas import tpu as pltpu
from jax.experimental.pallas import tpu_sc as plsc

_B, _S, _L, _D = 1024, 20, 4, 768
_VOCAB = 102
_T = _B * _S
_NC, _NS = 2, 16
_NW = _NC * _NS
_TPW = _T // _NW
_CHUNK = 32
_NCHUNK = _TPW // _CHUNK


def _body(ids_hbm, table_hbm, out_hbm, table_v, ids_v, obuf):
    wid = lax.axis_index("s") * _NC + lax.axis_index("c")
    base = wid * _TPW
    pltpu.sync_copy(table_hbm, table_v)
    pltpu.sync_copy(ids_hbm.at[pl.ds(base * _L, _TPW * _L)], ids_v)
    zero = jnp.zeros((16,), jnp.float32)
    for j in range(_D // 16):
        table_v[0, pl.ds(j * 16, 16)] = zero

    def chunk_body(c, carry):
        def grp_body(g, carry2):
            iv = ids_v[pl.ds((c * _CHUNK + g * 4) * _L, 16)]
            rows = [iv[k] for k in range(16)]

            @plsc.parallel_loop(0, _D // 16, unroll=4)
            def j_body(j):
                ds = pl.ds(j * 16, 16)
                for tt in range(4):
                    i0, i1, i2, i3 = rows[4 * tt:4 * tt + 4]
                    v = jnp.maximum(
                        jnp.maximum(table_v[i0, ds], table_v[i1, ds]),
                        jnp.maximum(table_v[i2, ds], table_v[i3, ds]))
                    obuf[g * 4 + tt, ds] = v
            return carry2

        lax.fori_loop(0, _CHUNK // 4, grp_body, 0)
        pltpu.sync_copy(obuf, out_hbm.at[pl.ds(base + c * _CHUNK, _CHUNK)])
        return carry

    lax.fori_loop(0, _NCHUNK, chunk_body, 0)


@jax.jit
def _glyph(ids_flat, table):
    mesh = plsc.VectorSubcoreMesh(core_axis_name="c", subcore_axis_name="s")
    f = pl.kernel(
        _body,
        out_type=jax.ShapeDtypeStruct((_T, _D), jnp.float32),
        mesh=mesh,
        scratch_types=[
            pltpu.VMEM((_VOCAB, _D), jnp.float32),
            pltpu.VMEM((_TPW * _L,), jnp.int32),
            pltpu.VMEM((_CHUNK, _D), jnp.float32),
        ],
    )
    return f(ids_flat, table)


def kernel(zixing_ids, table):
    ids_flat = zixing_ids.reshape(_T * _L)
    out = _glyph(ids_flat, table)
    return out.reshape(_B, _S, _D)


# speedup vs baseline: 1.9962x; 1.0706x over previous
import jax
import jax.numpy as jnp
from jax import lax
from jax.experimental import pallas as pl
from jax.experimental.pallas import tpu as pltpu
from jax.experimental.pallas import tpu_sc as plsc

_B, _S, _L, _D = 1024, 20, 4, 768
_VOCAB = 102
_T = _B * _S
_NC, _NS = 2, 16
_NW = _NC * _NS
_TPW = _T // _NW
_CHUNK = 16
_NCHUNK = _TPW // _CHUNK


def _body(ids_hbm, table_hbm, out_hbm, table_v, ids_v, obuf, sem0, sem1):
    wid = lax.axis_index("s") * _NC + lax.axis_index("c")
    base = wid * _TPW
    pltpu.sync_copy(table_hbm, table_v)
    pltpu.sync_copy(ids_hbm.at[pl.ds(base * _L, _TPW * _L)], ids_v)
    zero = jnp.zeros((16,), jnp.float32)
    for j in range(_D // 16):
        table_v[0, pl.ds(j * 16, 16)] = zero

    sems = (sem0, sem1)

    def pair_body(c2, carry):
        for b in range(2):
            c = c2 * 2 + b

            @pl.when(c2 > 0)
            def _wait():
                pltpu.make_async_copy(
                    obuf.at[b], out_hbm.at[pl.ds(base, _CHUNK)], sems[b]).wait()

            def grp_body(g, carry2):
                iv = ids_v[pl.ds((c * _CHUNK + g * 4) * _L, 16)]
                rows = [iv[k] for k in range(16)]

                @plsc.parallel_loop(0, _D // 16, unroll=4)
                def j_body(j):
                    ds = pl.ds(j * 16, 16)
                    for tt in range(4):
                        i0, i1, i2, i3 = rows[4 * tt:4 * tt + 4]
                        v = jnp.maximum(
                            jnp.maximum(table_v[i0, ds], table_v[i1, ds]),
                            jnp.maximum(table_v[i2, ds], table_v[i3, ds]))
                        obuf[b, g * 4 + tt, ds] = v
                return carry2

            lax.fori_loop(0, _CHUNK // 4, grp_body, 0)
            pltpu.async_copy(
                obuf.at[b], out_hbm.at[pl.ds(base + c * _CHUNK, _CHUNK)],
                sems[b])
        return carry

    lax.fori_loop(0, _NCHUNK // 2, pair_body, 0)
    for b in range(2):
        pltpu.make_async_copy(
            obuf.at[b], out_hbm.at[pl.ds(base, _CHUNK)], sems[b]).wait()


@jax.jit
def _glyph(ids_flat, table):
    mesh = plsc.VectorSubcoreMesh(core_axis_name="c", subcore_axis_name="s")
    f = pl.kernel(
        _body,
        out_type=jax.ShapeDtypeStruct((_T, _D), jnp.float32),
        mesh=mesh,
        scratch_types=[
            pltpu.VMEM((_VOCAB, _D), jnp.float32),
            pltpu.VMEM((_TPW * _L,), jnp.int32),
            pltpu.VMEM((2, _CHUNK, _D), jnp.float32),
            pltpu.SemaphoreType.DMA,
            pltpu.SemaphoreType.DMA,
        ],
    )
    return f(ids_flat, table)


def kernel(zixing_ids, table):
    ids_flat = zixing_ids.reshape(_T * _L)
    out = _glyph(ids_flat, table)
    return out.reshape(_B, _S, _D)


# speedup vs baseline: 2.8821x; 1.4438x over previous
import jax
import jax.numpy as jnp
from jax import lax
from jax.experimental import pallas as pl
from jax.experimental.pallas import tpu as pltpu
from jax.experimental.pallas import tpu_sc as plsc

_B, _S, _L, _D = 1024, 20, 4, 768
_VOCAB = 102
_T = _B * _S
_NC, _NS = 2, 16
_NW = _NC * _NS
_TPW = _T // _NW
_BPW = _B // _NW
_CHUNK = _S


def _body(ids_hbm, table_hbm, out_hbm, table_v, ids_v, obuf, sem0, sem1):
    wid = lax.axis_index("s") * _NC + lax.axis_index("c")
    base = wid * _TPW
    pltpu.sync_copy(table_hbm, table_v)
    pltpu.sync_copy(ids_hbm.at[pl.ds(base * _L, _TPW * _L)], ids_v)
    zero = jnp.zeros((16,), jnp.float32)
    for j in range(_D // 16):
        table_v[0, pl.ds(j * 16, 16)] = zero
    sems = (sem0, sem1)

    def pair_body(c2, carry):
        for b in range(2):
            c = c2 * 2 + b

            @pl.when(c2 > 0)
            def _wait():
                pltpu.make_async_copy(
                    obuf.at[b], out_hbm.at[wid * _BPW], sems[b]).wait()

            def grp_body(g, carry2):
                iv = ids_v[pl.ds((c * _CHUNK + g * 4) * _L, 16)]
                rows = [iv[k] for k in range(16)]

                @plsc.parallel_loop(0, _D // 16, unroll=4)
                def j_body(j):
                    ds = pl.ds(j * 16, 16)
                    for tt in range(4):
                        i0, i1, i2, i3 = rows[4 * tt:4 * tt + 4]
                        v = jnp.maximum(
                            jnp.maximum(table_v[i0, ds], table_v[i1, ds]),
                            jnp.maximum(table_v[i2, ds], table_v[i3, ds]))
                        obuf[b, g * 4 + tt, ds] = v
                return carry2

            lax.fori_loop(0, _CHUNK // 4, grp_body, 0)
            pltpu.async_copy(obuf.at[b], out_hbm.at[wid * _BPW + c], sems[b])
        return carry

    lax.fori_loop(0, _BPW // 2, pair_body, 0)
    for b in range(2):
        pltpu.make_async_copy(
            obuf.at[b], out_hbm.at[wid * _BPW], sems[b]).wait()


@jax.jit
def _glyph(ids_flat, table):
    mesh = plsc.VectorSubcoreMesh(core_axis_name="c", subcore_axis_name="s")
    f = pl.kernel(
        _body,
        out_type=jax.ShapeDtypeStruct((_B, _S, _D), jnp.float32),
        mesh=mesh,
        scratch_types=[
            pltpu.VMEM((_VOCAB, _D), jnp.float32),
            pltpu.VMEM((_TPW * _L,), jnp.int32),
            pltpu.VMEM((2, _CHUNK, _D), jnp.float32),
            pltpu.SemaphoreType.DMA,
            pltpu.SemaphoreType.DMA,
        ],
    )
    return f(ids_flat, table)


def kernel(zixing_ids, table):
    ids_flat = zixing_ids.reshape(_T * _L)
    return _glyph(ids_flat, table)
